# baseline (speedup 1.0000x reference)
import functools

import jax
import jax.numpy as jnp
from jax import lax
from jax.experimental import pallas as pl
from jax.experimental.pallas import tpu as pltpu
from jax.experimental.pallas import tpu_sc as plsc

B = 16384
D = 64
NC = 2
NS = 16
NW = NC * NS
BPW = B // NW
NCH = 4
CH = BPW // NCH
GPC = CH // 16

_mesh = plsc.VectorSubcoreMesh(core_axis_name="c", subcore_axis_name="s")


def _gmf_body(uidx_hbm, iidx_hbm, ut_hbm, it_hbm, w_hbm, b_hbm, out_hbm,
              uidx_v, iidx_v, urows, irows, w_v, b_v, out_v,
              sem0, sem1, sem2, sem3):
    c = lax.axis_index("c")
    s = lax.axis_index("s")
    wid = s * NC + c
    base = wid * BPW

    pltpu.sync_copy(uidx_hbm.at[wid], uidx_v)
    pltpu.sync_copy(iidx_hbm.at[wid], iidx_v)
    pltpu.sync_copy(w_hbm, w_v)
    pltpu.sync_copy(b_hbm, b_v)

    sems = [sem0, sem1, sem2, sem3]
    copies = []
    for j in range(NCH):
        copies.append((
            pltpu.async_copy(
                ut_hbm.at[uidx_v.at[j]], urows.at[pl.ds(j * CH, CH)], sems[j]),
            pltpu.async_copy(
                it_hbm.at[iidx_v.at[j]], irows.at[pl.ds(j * CH, CH)], sems[j]),
        ))

    w_chunks = [w_v[pl.ds(k * 16, 16)] for k in range(4)]
    lane = lax.iota(jnp.int32, 16)
    m15 = lane == 15
    lane_consts = [jnp.full((16,), r, jnp.int32) for r in range(16)]

    for j in range(NCH):
        copies[j][0].wait()
        copies[j][1].wait()

        def group_body(g, carry, j=j):
            rb = j * CH + g * 16
            view = out_v.at[pl.ds(rb, 16)]
            for r in range(16):
                row = rb + r
                acc = (urows[row, pl.ds(0, 16)] * irows[row, pl.ds(0, 16)]
                       * w_chunks[0])
                for k in range(1, 4):
                    acc = acc + (urows[row, pl.ds(k * 16, 16)]
                                 * irows[row, pl.ds(k * 16, 16)]
                                 * w_chunks[k])
                cum = plsc.cumsum(acc)
                plsc.store_scatter(view, [lane_consts[r]], cum, mask=m15)
            return carry

        lax.fori_loop(0, GPC, group_body, 0)

    b_vec = b_v[...]

    def sig_body(k, carry):
        v = out_v[pl.ds(k * 16, 16)] + b_vec
        out_v[pl.ds(k * 16, 16)] = 1.0 / (1.0 + jnp.exp(-v))
        return carry

    lax.fori_loop(0, BPW // 16, sig_body, 0)
    pltpu.sync_copy(out_v, out_hbm.at[pl.ds(base, BPW)])


_gmf_call = functools.partial(
    pl.kernel,
    out_type=jax.ShapeDtypeStruct((B,), jnp.float32),
    mesh=_mesh,
    compiler_params=pltpu.CompilerParams(
        needs_layout_passes=False, use_tc_tiling_on_sc=False),
    scratch_types=[
        pltpu.VMEM((NCH, CH), jnp.int32),
        pltpu.VMEM((NCH, CH), jnp.int32),
        pltpu.VMEM((BPW, D), jnp.float32),
        pltpu.VMEM((BPW, D), jnp.float32),
        pltpu.VMEM((D,), jnp.float32),
        pltpu.VMEM((16,), jnp.float32),
        pltpu.VMEM((BPW,), jnp.float32),
        pltpu.SemaphoreType.DMA,
        pltpu.SemaphoreType.DMA,
        pltpu.SemaphoreType.DMA,
        pltpu.SemaphoreType.DMA,
    ],
)(_gmf_body)


@jax.jit
def kernel(user_indices, item_indices, user_table, item_table, W, b):
    ui = user_indices.astype(jnp.int32).reshape(NW, NCH, CH)
    ii = item_indices.astype(jnp.int32).reshape(NW, NCH, CH)
    wf = jnp.reshape(W, (D,)).astype(jnp.float32)
    bb = jnp.broadcast_to(jnp.reshape(b, (1,)), (16,)).astype(jnp.float32)
    out = _gmf_call(ui, ii, user_table, item_table, wf, bb)
    return out.reshape(B, 1)


# speedup vs baseline: 2.4114x; 2.4114x over previous
import functools

import jax
import jax.numpy as jnp
from jax import lax
from jax.experimental import pallas as pl
from jax.experimental.pallas import tpu as pltpu
from jax.experimental.pallas import tpu_sc as plsc

B = 16384
D = 64
NU = 1000000
NI = 100000
NC = 2
NS = 16
NW = NC * NS
BPW = B // NW
SB = 512
SHIFT = 9
NSB_U = NU // SB
NSB_I = NI // SB
TAIL_U = NU - NSB_U * SB
TAIL_I = NI - NSB_I * SB
PER_U, REM_U = NSB_U // NW, NSB_U % NW
PER_I, REM_I = NSB_I // NW, NSB_I % NW
LCAP = 2048
NVEC = B // 16

_mesh = plsc.VectorSubcoreMesh(core_axis_name="c", subcore_axis_name="s")


def _extract_body(uidx_hbm, iidx_hbm, ut_hbm, it_hbm, utail_hbm, itail_hbm,
                  uflat_hbm, iflat_hbm,
                  idx_v, listb, listi, blb, blc, blockbuf, tailu, taili,
                  stage, semA, semB, semO):
    c = lax.axis_index("c")
    s = lax.axis_index("s")
    wid = s * NC + c
    lane = lax.iota(jnp.int32, 16)
    dvecs = [lane + 16 * k for k in range(4)]
    i32 = jnp.int32

    def _drain_one(_, z):
        pltpu.make_async_copy(
            uflat_hbm.at[pl.ds(0, D)], stage.at[0], semO).wait()
        return z

    def run_table(idx_hbm, tbl_hbm, tail_hbm, tailbuf, out_hbm,
                  nsb, per, rem, tail_w):
        lo = wid * per + jnp.minimum(wid, i32(rem))
        cnt = per + jnp.where(wid < rem, 1, 0).astype(i32)
        is_tail_owner = wid == NW - 1
        hi = lo + cnt + jnp.where(is_tail_owner, 1, 0).astype(i32)

        pltpu.sync_copy(idx_hbm, idx_v)

        def scan_body(v, off):
            idxv = idx_v[pl.ds(v * 16, 16)]
            bv = v * 16 + lane
            js = lax.shift_right_logical(idxv, SHIFT)
            m = (js >= lo) & (js < hi)
            plsc.store_compressed(listb.at[pl.ds(off, 16)], bv, mask=m)
            plsc.store_compressed(listi.at[pl.ds(off, 16)], idxv, mask=m)
            return off + plsc.all_reduce_population_count(m)[0]

        total = lax.fori_loop(0, NVEC, scan_body, i32(0))
        nlv = (total + 15) // 16

        def process_block(buf, jcur, mc):
            def cb_body(vi, boff):
                bs = listb[pl.ds(vi * 16, 16)]
                ids = listi[pl.ds(vi * 16, 16)]
                jv = lax.shift_right_logical(ids, SHIFT)
                m = (jv == jcur) & ((vi * 16 + lane) < total)
                plsc.store_compressed(blb.at[pl.ds(boff, 16)], bs, mask=m)
                plsc.store_compressed(
                    blc.at[pl.ds(boff, 16)], ids & (SB - 1), mask=m)
                return boff + plsc.all_reduce_population_count(m)[0]

            bc = lax.fori_loop(0, nlv, cb_body, i32(0))

            def ext_vec(vi, mc):
                bvec = blb[pl.ds(vi * 16, 16)]
                cvec = blc[pl.ds(vi * 16, 16)]
                nval = bc - vi * 16
                for k in range(16):
                    @pl.when(k < nval)
                    def _(k=k, mc=mc):
                        slot = (mc + k) & 31
                        colv = jnp.broadcast_to(cvec[k], (16,))
                        for q in range(4):
                            stage[slot, pl.ds(q * 16, 16)] = (
                                plsc.load_gather(buf, [dvecs[q], colv]))
                        pltpu.async_copy(
                            stage.at[slot],
                            out_hbm.at[pl.ds(bvec[k] * D, D)], semO)

                        @pl.when(slot == 31)
                        def _():
                            lax.fori_loop(0, 16, _drain_one, 0)

                return mc + jnp.minimum(nval, 16)

            nbv = lax.shift_right_logical(bc + 15, 4)
            return lax.fori_loop(0, nbv, ext_vec, mc)

        def fire(jblk, p):
            dst = blockbuf.at[0] if p == 0 else blockbuf.at[1]
            sem = semA if p == 0 else semB
            pltpu.async_copy(
                tbl_hbm.at[:, pl.ds(jblk * SB, SB)], dst, sem)

        def drain(p):
            dst = blockbuf.at[0] if p == 0 else blockbuf.at[1]
            sem = semA if p == 0 else semB
            pltpu.make_async_copy(
                tbl_hbm.at[:, pl.ds(0, SB)], dst, sem).wait()

        @pl.when(cnt > 0)
        def _():
            fire(lo, 0)

        @pl.when(cnt > 1)
        def _():
            fire(lo + 1, 1)

        def pair_body(t, mc):
            k0 = 2 * t

            def half(p, mc):
                k = k0 + p

                def work(mc):
                    drain(p)
                    mc = process_block(
                        blockbuf.at[0] if p == 0 else blockbuf.at[1],
                        lo + k, mc)

                    @pl.when(k + 2 < cnt)
                    def _():
                        fire(lo + k + 2, p)

                    return mc

                return lax.cond(k < cnt, work, lambda mc: mc, mc)

            mc = half(0, mc)
            return half(1, mc)

        npair = (cnt + 1) // 2
        mc = lax.fori_loop(0, npair, pair_body, i32(0))

        if tail_w:
            @pl.when(is_tail_owner)
            def _():
                pltpu.sync_copy(tail_hbm, tailbuf)

            mc = lax.cond(
                is_tail_owner,
                lambda mc: process_block(tailbuf, i32(nsb), mc),
                lambda mc: mc, mc)

        ndrained = 16 * lax.shift_right_logical(mc, 5)
        lax.fori_loop(0, mc - ndrained, _drain_one, i32(0))

    run_table(uidx_hbm, ut_hbm, utail_hbm, tailu, uflat_hbm,
              NSB_U, PER_U, REM_U, TAIL_U)
    run_table(iidx_hbm, it_hbm, itail_hbm, taili, iflat_hbm,
              NSB_I, PER_I, REM_I, TAIL_I)


_extract_call = functools.partial(
    pl.kernel,
    out_type=(jax.ShapeDtypeStruct((B * D,), jnp.float32),
              jax.ShapeDtypeStruct((B * D,), jnp.float32)),
    mesh=_mesh,
    compiler_params=pltpu.CompilerParams(
        needs_layout_passes=False, use_tc_tiling_on_sc=True),
    scratch_types=[
        pltpu.VMEM((B,), jnp.int32),
        pltpu.VMEM((LCAP,), jnp.int32),
        pltpu.VMEM((LCAP,), jnp.int32),
        pltpu.VMEM((272,), jnp.int32),
        pltpu.VMEM((272,), jnp.int32),
        pltpu.VMEM((2, D, SB), jnp.float32),
        pltpu.VMEM((D, TAIL_U), jnp.float32),
        pltpu.VMEM((D, TAIL_I), jnp.float32),
        pltpu.VMEM((32, D), jnp.float32),
        pltpu.SemaphoreType.DMA,
        pltpu.SemaphoreType.DMA,
        pltpu.SemaphoreType.DMA,
    ],
)(_extract_body)


def _compute_body(uflat_hbm, iflat_hbm, w_hbm, b_hbm, out_hbm,
                  urows, irows, w_v, b_v, out_v):
    c = lax.axis_index("c")
    s = lax.axis_index("s")
    wid = s * NC + c
    base = wid * BPW

    pltpu.sync_copy(uflat_hbm.at[pl.ds(base * D, BPW * D)], urows)
    pltpu.sync_copy(iflat_hbm.at[pl.ds(base * D, BPW * D)], irows)
    pltpu.sync_copy(w_hbm, w_v)
    pltpu.sync_copy(b_hbm, b_v)

    w_chunks = [w_v[pl.ds(k * 16, 16)] for k in range(4)]
    lane = lax.iota(jnp.int32, 16)
    m15 = lane == 15
    lane_consts = [jnp.full((16,), r, jnp.int32) for r in range(16)]

    def group_body(g, carry):
        rb = g * 16
        view = out_v.at[pl.ds(rb, 16)]
        for r in range(16):
            row64 = (rb + r) * D
            acc = (urows[pl.ds(row64, 16)] * irows[pl.ds(row64, 16)]
                   * w_chunks[0])
            for k in range(1, 4):
                acc = acc + (urows[pl.ds(row64 + k * 16, 16)]
                             * irows[pl.ds(row64 + k * 16, 16)]
                             * w_chunks[k])
            cum = plsc.cumsum(acc)
            plsc.store_scatter(view, [lane_consts[r]], cum, mask=m15)
        return carry

    lax.fori_loop(0, BPW // 16, group_body, 0)

    b_vec = b_v[...]

    def sig_body(k, carry):
        v = out_v[pl.ds(k * 16, 16)] + b_vec
        out_v[pl.ds(k * 16, 16)] = 1.0 / (1.0 + jnp.exp(-v))
        return carry

    lax.fori_loop(0, BPW // 16, sig_body, 0)
    pltpu.sync_copy(out_v, out_hbm.at[pl.ds(base, BPW)])


_compute_call = functools.partial(
    pl.kernel,
    out_type=jax.ShapeDtypeStruct((B,), jnp.float32),
    mesh=_mesh,
    compiler_params=pltpu.CompilerParams(
        needs_layout_passes=False, use_tc_tiling_on_sc=True),
    scratch_types=[
        pltpu.VMEM((BPW * D,), jnp.float32),
        pltpu.VMEM((BPW * D,), jnp.float32),
        pltpu.VMEM((D,), jnp.float32),
        pltpu.VMEM((16,), jnp.float32),
        pltpu.VMEM((BPW,), jnp.float32),
    ],
)(_compute_body)


@jax.jit
def kernel(user_indices, item_indices, user_table, item_table, W, b):
    ui = user_indices.astype(jnp.int32)
    ii = item_indices.astype(jnp.int32)
    ut_t = user_table.T
    it_t = item_table.T
    ut_tail = user_table[NSB_U * SB:].T
    it_tail = item_table[NSB_I * SB:].T
    wf = jnp.reshape(W, (D,)).astype(jnp.float32)
    bb = jnp.broadcast_to(jnp.reshape(b, (1,)), (16,)).astype(jnp.float32)
    uflat, iflat = _extract_call(ui, ii, ut_t, it_t, ut_tail, it_tail)
    out = _compute_call(uflat, iflat, wf, bb)
    return out.reshape(B, 1)


# speedup vs baseline: 2.7892x; 1.1567x over previous
import functools

import jax
import jax.numpy as jnp
from jax import lax
from jax.experimental import pallas as pl
from jax.experimental.pallas import tpu as pltpu
from jax.experimental.pallas import tpu_sc as plsc

B = 16384
D = 64
NU = 1000000
NI = 100000
NC = 2
NS = 16
NW = NC * NS
BPW = B // NW
SB = 512
SHIFT = 9
NSB_U = NU // SB
NSB_I = NI // SB
TAIL_U = NU - NSB_U * SB
TAIL_I = NI - NSB_I * SB
PER_U, REM_U = NSB_U // NW, NSB_U % NW
PER_I, REM_I = NSB_I // NW, NSB_I % NW
LCAP = 2048
CAPQ = 512
NVEC = B // 16

_mesh = plsc.VectorSubcoreMesh(core_axis_name="c", subcore_axis_name="s")


def _extract_body(uidx_hbm, iidx_hbm, ut_hbm, it_hbm, utail_hbm, itail_hbm,
                  uflat_hbm, iflat_hbm,
                  idx_v, listb, listi, l4b, l4c, blb, blc, blockbuf,
                  tailu, taili, stage, semA, semB, semO):
    c = lax.axis_index("c")
    s = lax.axis_index("s")
    wid = s * NC + c
    lane = lax.iota(jnp.int32, 16)
    dvecs = [lane + 16 * k for k in range(4)]
    i32 = jnp.int32

    def _drain_one(_, z):
        pltpu.make_async_copy(
            uflat_hbm.at[pl.ds(0, D)], stage.at[0], semO).wait()
        return z

    def run_table(idx_hbm, tbl_hbm, tail_hbm, tailbuf, out_hbm,
                  nsb, per, rem, tail_w):
        lo = wid * per + jnp.minimum(wid, i32(rem))
        cnt = per + jnp.where(wid < rem, 1, 0).astype(i32)
        is_tail_owner = wid == NW - 1
        hi = lo + cnt + jnp.where(is_tail_owner, 1, 0).astype(i32)

        pltpu.sync_copy(idx_hbm, idx_v)

        def scan_body(v, off):
            idxv = idx_v[pl.ds(v * 16, 16)]
            bv = v * 16 + lane
            js = lax.shift_right_logical(idxv, SHIFT)
            m = (js >= lo) & (js < hi)
            plsc.store_compressed(listb.at[pl.ds(off, 16)], bv, mask=m)
            plsc.store_compressed(listi.at[pl.ds(off, 16)], idxv, mask=m)
            return off + plsc.all_reduce_population_count(m)[0]

        total = lax.fori_loop(0, NVEC, scan_body, i32(0))
        nlv = (total + 15) // 16

        lob = lo * SB

        def split_body(vi, offs):
            bs = listb[pl.ds(vi * 16, 16)]
            rel = listi[pl.ds(vi * 16, 16)] - lob
            q = lax.shift_right_logical(rel, SHIFT + 4)
            valid = (vi * 16 + lane) < total
            new = []
            for qq in range(4):
                mq = valid & (q == qq)
                plsc.store_compressed(
                    l4b.at[pl.ds(qq * CAPQ + offs[qq], 16)], bs, mask=mq)
                plsc.store_compressed(
                    l4c.at[pl.ds(qq * CAPQ + offs[qq], 16)], rel, mask=mq)
                new.append(offs[qq]
                           + plsc.all_reduce_population_count(mq)[0])
            return tuple(new)

        totq = lax.fori_loop(
            0, nlv, split_body, (i32(0), i32(0), i32(0), i32(0)))

        def process_block(buf, jloc, mc):
            q = lax.shift_right_logical(jloc, 4)
            qb = q * CAPQ
            tq = jnp.where(q == 0, totq[0],
                           jnp.where(q == 1, totq[1],
                                     jnp.where(q == 2, totq[2], totq[3])))
            nlvq = (tq + 15) // 16

            def cb_body(vi, boff):
                bs = l4b[pl.ds(qb + vi * 16, 16)]
                rel = l4c[pl.ds(qb + vi * 16, 16)]
                jv = lax.shift_right_logical(rel, SHIFT)
                m = (jv == jloc) & ((vi * 16 + lane) < tq)
                plsc.store_compressed(blb.at[pl.ds(boff, 16)], bs, mask=m)
                plsc.store_compressed(
                    blc.at[pl.ds(boff, 16)], rel & (SB - 1), mask=m)
                return boff + plsc.all_reduce_population_count(m)[0]

            bc = lax.fori_loop(0, nlvq, cb_body, i32(0))

            def ext_body(k, mc):
                brow = blb[pl.ds(k, 16)][0]
                col = blc[pl.ds(k, 16)][0]
                slot = mc & 31
                colv = jnp.broadcast_to(col, (16,))
                for q in range(4):
                    stage[slot, pl.ds(q * 16, 16)] = (
                        plsc.load_gather(buf, [dvecs[q], colv]))
                pltpu.async_copy(
                    stage.at[slot], out_hbm.at[pl.ds(brow * D, D)], semO)

                @pl.when(slot == 31)
                def _():
                    lax.fori_loop(0, 16, _drain_one, 0)

                return mc + 1

            return lax.fori_loop(0, bc, ext_body, mc)

        def fire(jblk, p):
            dst = blockbuf.at[0] if p == 0 else blockbuf.at[1]
            sem = semA if p == 0 else semB
            pltpu.async_copy(
                tbl_hbm.at[:, pl.ds(jblk * SB, SB)], dst, sem)

        def drain(p):
            dst = blockbuf.at[0] if p == 0 else blockbuf.at[1]
            sem = semA if p == 0 else semB
            pltpu.make_async_copy(
                tbl_hbm.at[:, pl.ds(0, SB)], dst, sem).wait()

        @pl.when(cnt > 0)
        def _():
            fire(lo, 0)

        @pl.when(cnt > 1)
        def _():
            fire(lo + 1, 1)

        def pair_body(t, mc):
            k0 = 2 * t

            def half(p, mc):
                k = k0 + p

                def work(mc):
                    drain(p)
                    mc = process_block(
                        blockbuf.at[0] if p == 0 else blockbuf.at[1],
                        k, mc)

                    @pl.when(k + 2 < cnt)
                    def _():
                        fire(lo + k + 2, p)

                    return mc

                return lax.cond(k < cnt, work, lambda mc: mc, mc)

            mc = half(0, mc)
            return half(1, mc)

        npair = (cnt + 1) // 2
        mc = lax.fori_loop(0, npair, pair_body, i32(0))

        if tail_w:
            @pl.when(is_tail_owner)
            def _():
                pltpu.sync_copy(tail_hbm, tailbuf)

            mc = lax.cond(
                is_tail_owner,
                lambda mc: process_block(tailbuf, cnt, mc),
                lambda mc: mc, mc)

        ndrained = 16 * lax.shift_right_logical(mc, 5)
        lax.fori_loop(0, mc - ndrained, _drain_one, i32(0))

    run_table(uidx_hbm, ut_hbm, utail_hbm, tailu, uflat_hbm,
              NSB_U, PER_U, REM_U, TAIL_U)
    run_table(iidx_hbm, it_hbm, itail_hbm, taili, iflat_hbm,
              NSB_I, PER_I, REM_I, TAIL_I)


_extract_call = functools.partial(
    pl.kernel,
    out_type=(jax.ShapeDtypeStruct((B * D,), jnp.float32),
              jax.ShapeDtypeStruct((B * D,), jnp.float32)),
    mesh=_mesh,
    compiler_params=pltpu.CompilerParams(
        needs_layout_passes=False, use_tc_tiling_on_sc=True),
    scratch_types=[
        pltpu.VMEM((B,), jnp.int32),
        pltpu.VMEM((LCAP,), jnp.int32),
        pltpu.VMEM((LCAP,), jnp.int32),
        pltpu.VMEM((4 * CAPQ + 16,), jnp.int32),
        pltpu.VMEM((4 * CAPQ + 16,), jnp.int32),
        pltpu.VMEM((272,), jnp.int32),
        pltpu.VMEM((272,), jnp.int32),
        pltpu.VMEM((2, D, SB), jnp.float32),
        pltpu.VMEM((D, TAIL_U), jnp.float32),
        pltpu.VMEM((D, TAIL_I), jnp.float32),
        pltpu.VMEM((32, D), jnp.float32),
        pltpu.SemaphoreType.DMA,
        pltpu.SemaphoreType.DMA,
        pltpu.SemaphoreType.DMA,
    ],
)(_extract_body)


def _compute_body(uflat_hbm, iflat_hbm, w_hbm, b_hbm, out_hbm,
                  urows, irows, w_v, b_v, out_v):
    c = lax.axis_index("c")
    s = lax.axis_index("s")
    wid = s * NC + c
    base = wid * BPW

    pltpu.sync_copy(uflat_hbm.at[pl.ds(base * D, BPW * D)], urows)
    pltpu.sync_copy(iflat_hbm.at[pl.ds(base * D, BPW * D)], irows)
    pltpu.sync_copy(w_hbm, w_v)
    pltpu.sync_copy(b_hbm, b_v)

    w_chunks = [w_v[pl.ds(k * 16, 16)] for k in range(4)]
    lane = lax.iota(jnp.int32, 16)
    m15 = lane == 15
    lane_consts = [jnp.full((16,), r, jnp.int32) for r in range(16)]

    def group_body(g, carry):
        rb = g * 16
        view = out_v.at[pl.ds(rb, 16)]
        for r in range(16):
            row64 = (rb + r) * D
            acc = (urows[pl.ds(row64, 16)] * irows[pl.ds(row64, 16)]
                   * w_chunks[0])
            for k in range(1, 4):
                acc = acc + (urows[pl.ds(row64 + k * 16, 16)]
                             * irows[pl.ds(row64 + k * 16, 16)]
                             * w_chunks[k])
            cum = plsc.cumsum(acc)
            plsc.store_scatter(view, [lane_consts[r]], cum, mask=m15)
        return carry

    lax.fori_loop(0, BPW // 16, group_body, 0)

    b_vec = b_v[...]

    def sig_body(k, carry):
        v = out_v[pl.ds(k * 16, 16)] + b_vec
        out_v[pl.ds(k * 16, 16)] = 1.0 / (1.0 + jnp.exp(-v))
        return carry

    lax.fori_loop(0, BPW // 16, sig_body, 0)
    pltpu.sync_copy(out_v, out_hbm.at[pl.ds(base, BPW)])


_compute_call = functools.partial(
    pl.kernel,
    out_type=jax.ShapeDtypeStruct((B,), jnp.float32),
    mesh=_mesh,
    compiler_params=pltpu.CompilerParams(
        needs_layout_passes=False, use_tc_tiling_on_sc=True),
    scratch_types=[
        pltpu.VMEM((BPW * D,), jnp.float32),
        pltpu.VMEM((BPW * D,), jnp.float32),
        pltpu.VMEM((D,), jnp.float32),
        pltpu.VMEM((16,), jnp.float32),
        pltpu.VMEM((BPW,), jnp.float32),
    ],
)(_compute_body)


@jax.jit
def kernel(user_indices, item_indices, user_table, item_table, W, b):
    ui = user_indices.astype(jnp.int32)
    ii = item_indices.astype(jnp.int32)
    ut_t = user_table.T
    it_t = item_table.T
    ut_tail = user_table[NSB_U * SB:].T
    it_tail = item_table[NSB_I * SB:].T
    wf = jnp.reshape(W, (D,)).astype(jnp.float32)
    bb = jnp.broadcast_to(jnp.reshape(b, (1,)), (16,)).astype(jnp.float32)
    uflat, iflat = _extract_call(ui, ii, ut_t, it_t, ut_tail, it_tail)
    out = _compute_call(uflat, iflat, wf, bb)
    return out.reshape(B, 1)


# speedup vs baseline: 2.8131x; 1.0086x over previous
import functools

import jax
import jax.numpy as jnp
from jax import lax
from jax.experimental import pallas as pl
from jax.experimental.pallas import tpu as pltpu
from jax.experimental.pallas import tpu_sc as plsc

B = 16384
D = 64
NU = 1000000
NI = 100000
NC = 2
NS = 16
NW = NC * NS
BPW = B // NW
SB = 512
SHIFT = 9
NSB_U = NU // SB
NSB_I = NI // SB
TAIL_U = NU - NSB_U * SB
TAIL_I = NI - NSB_I * SB
PER_U, REM_U = NSB_U // NW, NSB_U % NW
PER_I, REM_I = NSB_I // NW, NSB_I % NW
LCAP = 2048
NVEC = B // 16

_mesh = plsc.VectorSubcoreMesh(core_axis_name="c", subcore_axis_name="s")


def _extract_body(uidx_hbm, iidx_hbm, ut_hbm, it_hbm, utail_hbm, itail_hbm,
                  uflat_hbm, iflat_hbm,
                  idx_v, listb, listi, blb, blc, blockbuf, tailu, taili,
                  stage, semA, semB, semO):
    c = lax.axis_index("c")
    s = lax.axis_index("s")
    wid = s * NC + c
    lane = lax.iota(jnp.int32, 16)
    dvecs = [lane + 16 * k for k in range(4)]
    i32 = jnp.int32

    def _drain_one(_, z):
        pltpu.make_async_copy(
            uflat_hbm.at[pl.ds(0, D)], stage.at[0], semO).wait()
        return z

    def run_table(idx_hbm, tbl_hbm, tail_hbm, tailbuf, out_hbm,
                  nsb, per, rem, tail_w):
        lo = wid * per + jnp.minimum(wid, i32(rem))
        cnt = per + jnp.where(wid < rem, 1, 0).astype(i32)
        is_tail_owner = wid == NW - 1
        hi = lo + cnt + jnp.where(is_tail_owner, 1, 0).astype(i32)

        pltpu.sync_copy(idx_hbm, idx_v)

        def scan_body(v, off):
            idxv = idx_v[pl.ds(v * 16, 16)]
            bv = v * 16 + lane
            js = lax.shift_right_logical(idxv, SHIFT)
            m = (js >= lo) & (js < hi)
            plsc.store_compressed(listb.at[pl.ds(off, 16)], bv, mask=m)
            plsc.store_compressed(listi.at[pl.ds(off, 16)], idxv, mask=m)
            return off + plsc.all_reduce_population_count(m)[0]

        total = lax.fori_loop(0, NVEC, scan_body, i32(0))
        nlv = (total + 15) // 16

        def process_block(buf, jcur, mc):
            def cb_body(vi, boff):
                bs = listb[pl.ds(vi * 16, 16)]
                ids = listi[pl.ds(vi * 16, 16)]
                jv = lax.shift_right_logical(ids, SHIFT)
                m = (jv == jcur) & ((vi * 16 + lane) < total)
                plsc.store_compressed(blb.at[pl.ds(boff, 16)], bs, mask=m)
                plsc.store_compressed(
                    blc.at[pl.ds(boff, 16)], ids & (SB - 1), mask=m)
                return boff + plsc.all_reduce_population_count(m)[0]

            bc = lax.fori_loop(0, nlv, cb_body, i32(0))

            def ext_body(k, mc):
                brow = blb[pl.ds(k, 16)][0]
                col = blc[pl.ds(k, 16)][0]
                slot = mc & 31
                colv = jnp.broadcast_to(col, (16,))
                for q in range(4):
                    stage[slot, pl.ds(q * 16, 16)] = (
                        plsc.load_gather(buf, [dvecs[q], colv]))
                pltpu.async_copy(
                    stage.at[slot], out_hbm.at[pl.ds(brow * D, D)], semO)

                @pl.when(slot == 31)
                def _():
                    lax.fori_loop(0, 16, _drain_one, 0)

                return mc + 1

            return lax.fori_loop(0, bc, ext_body, mc)

        def fire(jblk, p):
            dst = blockbuf.at[0] if p == 0 else blockbuf.at[1]
            sem = semA if p == 0 else semB
            for seg in range(8):
                pltpu.async_copy(
                    tbl_hbm.at[pl.ds(8 * seg, 8), pl.ds(jblk * SB, SB)],
                    dst.at[pl.ds(8 * seg, 8)], sem)

        def drain(p):
            dst = blockbuf.at[0] if p == 0 else blockbuf.at[1]
            sem = semA if p == 0 else semB
            pltpu.make_async_copy(
                tbl_hbm.at[:, pl.ds(0, SB)], dst, sem).wait()

        @pl.when(cnt > 0)
        def _():
            fire(lo, 0)

        @pl.when(cnt > 1)
        def _():
            fire(lo + 1, 1)

        def pair_body(t, mc):
            k0 = 2 * t

            def half(p, mc):
                k = k0 + p

                def work(mc):
                    drain(p)
                    mc = process_block(
                        blockbuf.at[0] if p == 0 else blockbuf.at[1],
                        lo + k, mc)

                    @pl.when(k + 2 < cnt)
                    def _():
                        fire(lo + k + 2, p)

                    return mc

                return lax.cond(k < cnt, work, lambda mc: mc, mc)

            mc = half(0, mc)
            return half(1, mc)

        npair = (cnt + 1) // 2
        mc = lax.fori_loop(0, npair, pair_body, i32(0))

        if tail_w:
            @pl.when(is_tail_owner)
            def _():
                pltpu.sync_copy(tail_hbm, tailbuf)

            mc = lax.cond(
                is_tail_owner,
                lambda mc: process_block(tailbuf, i32(nsb), mc),
                lambda mc: mc, mc)

        ndrained = 16 * lax.shift_right_logical(mc, 5)
        lax.fori_loop(0, mc - ndrained, _drain_one, i32(0))

    run_table(uidx_hbm, ut_hbm, utail_hbm, tailu, uflat_hbm,
              NSB_U, PER_U, REM_U, TAIL_U)
    run_table(iidx_hbm, it_hbm, itail_hbm, taili, iflat_hbm,
              NSB_I, PER_I, REM_I, TAIL_I)


_extract_call = functools.partial(
    pl.kernel,
    out_type=(jax.ShapeDtypeStruct((B * D,), jnp.float32),
              jax.ShapeDtypeStruct((B * D,), jnp.float32)),
    mesh=_mesh,
    compiler_params=pltpu.CompilerParams(
        needs_layout_passes=False, use_tc_tiling_on_sc=True),
    scratch_types=[
        pltpu.VMEM((B,), jnp.int32),
        pltpu.VMEM((LCAP,), jnp.int32),
        pltpu.VMEM((LCAP,), jnp.int32),
        pltpu.VMEM((272,), jnp.int32),
        pltpu.VMEM((272,), jnp.int32),
        pltpu.VMEM((2, D, SB), jnp.float32),
        pltpu.VMEM((D, TAIL_U), jnp.float32),
        pltpu.VMEM((D, TAIL_I), jnp.float32),
        pltpu.VMEM((32, D), jnp.float32),
        pltpu.SemaphoreType.DMA,
        pltpu.SemaphoreType.DMA,
        pltpu.SemaphoreType.DMA,
    ],
)(_extract_body)


def _compute_body(uflat_hbm, iflat_hbm, w_hbm, b_hbm, out_hbm,
                  urows, irows, w_v, b_v, out_v):
    c = lax.axis_index("c")
    s = lax.axis_index("s")
    wid = s * NC + c
    base = wid * BPW

    pltpu.sync_copy(uflat_hbm.at[pl.ds(base * D, BPW * D)], urows)
    pltpu.sync_copy(iflat_hbm.at[pl.ds(base * D, BPW * D)], irows)
    pltpu.sync_copy(w_hbm, w_v)
    pltpu.sync_copy(b_hbm, b_v)

    w_chunks = [w_v[pl.ds(k * 16, 16)] for k in range(4)]
    lane = lax.iota(jnp.int32, 16)
    m15 = lane == 15
    lane_consts = [jnp.full((16,), r, jnp.int32) for r in range(16)]

    def group_body(g, carry):
        rb = g * 16
        view = out_v.at[pl.ds(rb, 16)]
        for r in range(16):
            row64 = (rb + r) * D
            acc = (urows[pl.ds(row64, 16)] * irows[pl.ds(row64, 16)]
                   * w_chunks[0])
            for k in range(1, 4):
                acc = acc + (urows[pl.ds(row64 + k * 16, 16)]
                             * irows[pl.ds(row64 + k * 16, 16)]
                             * w_chunks[k])
            cum = plsc.cumsum(acc)
            plsc.store_scatter(view, [lane_consts[r]], cum, mask=m15)
        return carry

    lax.fori_loop(0, BPW // 16, group_body, 0)

    b_vec = b_v[...]

    def sig_body(k, carry):
        v = out_v[pl.ds(k * 16, 16)] + b_vec
        out_v[pl.ds(k * 16, 16)] = 1.0 / (1.0 + jnp.exp(-v))
        return carry

    lax.fori_loop(0, BPW // 16, sig_body, 0)
    pltpu.sync_copy(out_v, out_hbm.at[pl.ds(base, BPW)])


_compute_call = functools.partial(
    pl.kernel,
    out_type=jax.ShapeDtypeStruct((B,), jnp.float32),
    mesh=_mesh,
    compiler_params=pltpu.CompilerParams(
        needs_layout_passes=False, use_tc_tiling_on_sc=True),
    scratch_types=[
        pltpu.VMEM((BPW * D,), jnp.float32),
        pltpu.VMEM((BPW * D,), jnp.float32),
        pltpu.VMEM((D,), jnp.float32),
        pltpu.VMEM((16,), jnp.float32),
        pltpu.VMEM((BPW,), jnp.float32),
    ],
)(_compute_body)


@jax.jit
def kernel(user_indices, item_indices, user_table, item_table, W, b):
    ui = user_indices.astype(jnp.int32)
    ii = item_indices.astype(jnp.int32)
    ut_t = user_table.T
    it_t = item_table.T
    ut_tail = user_table[NSB_U * SB:].T
    it_tail = item_table[NSB_I * SB:].T
    wf = jnp.reshape(W, (D,)).astype(jnp.float32)
    bb = jnp.broadcast_to(jnp.reshape(b, (1,)), (16,)).astype(jnp.float32)
    uflat, iflat = _extract_call(ui, ii, ut_t, it_t, ut_tail, it_tail)
    out = _compute_call(uflat, iflat, wf, bb)
    return out.reshape(B, 1)


# speedup vs baseline: 2.9351x; 1.0434x over previous
import functools

import jax
import jax.numpy as jnp
from jax import lax
from jax.experimental import pallas as pl
from jax.experimental.pallas import tpu as pltpu
from jax.experimental.pallas import tpu_sc as plsc

B = 16384
D = 64
NU = 1000000
NI = 100000
NC = 2
NS = 16
NW = NC * NS
BPW = B // NW
SB = 512
SHIFT = 9
NSB_U = NU // SB
NSB_I = NI // SB
TAIL_U = NU - NSB_U * SB
TAIL_I = NI - NSB_I * SB
PER_U, REM_U = NSB_U // NW, NSB_U % NW
PER_I, REM_I = NSB_I // NW, NSB_I % NW
LCAP = 1024
NVEC = B // 16

_mesh = plsc.VectorSubcoreMesh(core_axis_name="c", subcore_axis_name="s")


def _extract_body(uidx_hbm, iidx_hbm, ut_hbm, it_hbm, utail_hbm, itail_hbm,
                  uflat_hbm, iflat_hbm,
                  idx_v, idx2_v, ulistb, ulisti, ilistb, ilisti,
                  blb, blc, blockbuf, tailu, taili,
                  stage, semA, semB, semO):
    c = lax.axis_index("c")
    s = lax.axis_index("s")
    wid = s * NC + c
    lane = lax.iota(jnp.int32, 16)
    dvecs = [lane + 16 * k for k in range(4)]
    i32 = jnp.int32

    def _drain_one(_, z):
        pltpu.make_async_copy(
            uflat_hbm.at[pl.ds(0, D)], stage.at[0], semO).wait()
        return z

    def ranges(per, rem):
        lo = wid * per + jnp.minimum(wid, i32(rem))
        cnt = per + jnp.where(wid < rem, 1, 0).astype(i32)
        hi = lo + cnt + jnp.where(wid == NW - 1, 1, 0).astype(i32)
        return lo, cnt, hi

    lo_u, cnt_u, hi_u = ranges(PER_U, REM_U)
    lo_i, cnt_i, hi_i = ranges(PER_I, REM_I)

    pltpu.sync_copy(uidx_hbm, idx_v)
    pltpu.sync_copy(iidx_hbm, idx2_v)

    def scan_body(v, offs):
        ou, oi = offs
        bv = v * 16 + lane
        uv = idx_v[pl.ds(v * 16, 16)]
        iv = idx2_v[pl.ds(v * 16, 16)]
        ju = lax.shift_right_logical(uv, SHIFT)
        ji = lax.shift_right_logical(iv, SHIFT)
        mu = (ju >= lo_u) & (ju < hi_u)
        mi = (ji >= lo_i) & (ji < hi_i)
        plsc.store_compressed(ulistb.at[pl.ds(ou, 16)], bv, mask=mu)
        plsc.store_compressed(ulisti.at[pl.ds(ou, 16)], uv, mask=mu)
        plsc.store_compressed(ilistb.at[pl.ds(oi, 16)], bv, mask=mi)
        plsc.store_compressed(ilisti.at[pl.ds(oi, 16)], iv, mask=mi)
        return (ou + plsc.all_reduce_population_count(mu)[0],
                oi + plsc.all_reduce_population_count(mi)[0])

    tot_u, tot_i = lax.fori_loop(0, NVEC, scan_body, (i32(0), i32(0)))

    def run_table(listb, listi, total, lo, cnt, tbl_hbm, tail_hbm, tailbuf,
                  out_hbm, nsb, tail_w):
        is_tail_owner = wid == NW - 1
        nlv = (total + 15) // 16

        def process_block(buf, jcur, mc, tail_cols=None):
            def cb_body(vi, boff):
                bs = listb[pl.ds(vi * 16, 16)]
                ids = listi[pl.ds(vi * 16, 16)]
                jv = lax.shift_right_logical(ids, SHIFT)
                m = (jv == jcur) & ((vi * 16 + lane) < total)
                plsc.store_compressed(blb.at[pl.ds(boff, 16)], bs, mask=m)
                plsc.store_compressed(
                    blc.at[pl.ds(boff, 16)], ids & (SB - 1), mask=m)
                return boff + plsc.all_reduce_population_count(m)[0]

            bc = lax.fori_loop(0, nlv, cb_body, i32(0))

            def ext_body(k, mc):
                brow = blb[pl.ds(k, 16)][0]
                col = blc[pl.ds(k, 16)][0]
                slot = mc & 31
                colv = jnp.broadcast_to(col, (16,))
                for q in range(4):
                    if tail_cols is None:
                        vals = plsc.load_gather(buf, [dvecs[q], colv])
                    else:
                        vals = plsc.load_gather(
                            buf, [dvecs[q] * tail_cols + colv])
                    stage[slot, pl.ds(q * 16, 16)] = vals
                pltpu.async_copy(
                    stage.at[slot], out_hbm.at[pl.ds(brow * D, D)], semO)

                @pl.when(slot == 31)
                def _():
                    lax.fori_loop(0, 16, _drain_one, 0)

                return mc + 1

            return lax.fori_loop(0, bc, ext_body, mc)

        def fire(jblk, p):
            dst = blockbuf.at[0] if p == 0 else blockbuf.at[1]
            sem = semA if p == 0 else semB
            for seg in range(8):
                pltpu.async_copy(
                    tbl_hbm.at[pl.ds(8 * seg, 8), pl.ds(jblk * SB, SB)],
                    dst.at[pl.ds(8 * seg, 8)], sem)

        def drain(p):
            dst = blockbuf.at[0] if p == 0 else blockbuf.at[1]
            sem = semA if p == 0 else semB
            pltpu.make_async_copy(
                tbl_hbm.at[:, pl.ds(0, SB)], dst, sem).wait()

        @pl.when(cnt > 0)
        def _():
            fire(lo, 0)

        @pl.when(cnt > 1)
        def _():
            fire(lo + 1, 1)

        def pair_body(t, mc):
            k0 = 2 * t

            def half(p, mc):
                k = k0 + p

                def work(mc):
                    drain(p)
                    mc = process_block(
                        blockbuf.at[0] if p == 0 else blockbuf.at[1],
                        lo + k, mc)

                    @pl.when(k + 2 < cnt)
                    def _():
                        fire(lo + k + 2, p)

                    return mc

                return lax.cond(k < cnt, work, lambda mc: mc, mc)

            mc = half(0, mc)
            return half(1, mc)

        npair = (cnt + 1) // 2
        mc = lax.fori_loop(0, npair, pair_body, i32(0))

        if tail_w:
            @pl.when(is_tail_owner)
            def _():
                pltpu.sync_copy(tail_hbm, tailbuf)

            mc = lax.cond(
                is_tail_owner,
                lambda mc: process_block(tailbuf, i32(nsb), mc,
                                         tail_cols=tail_w),
                lambda mc: mc, mc)

        ndrained = 16 * lax.shift_right_logical(mc, 5)
        lax.fori_loop(0, mc - ndrained, _drain_one, i32(0))

    run_table(ulistb, ulisti, tot_u, lo_u, cnt_u, ut_hbm, utail_hbm, tailu,
              uflat_hbm, NSB_U, TAIL_U)
    run_table(ilistb, ilisti, tot_i, lo_i, cnt_i, it_hbm, itail_hbm, taili,
              iflat_hbm, NSB_I, TAIL_I)


_extract_call = functools.partial(
    pl.kernel,
    out_type=(jax.ShapeDtypeStruct((B * D,), jnp.float32),
              jax.ShapeDtypeStruct((B * D,), jnp.float32)),
    mesh=_mesh,
    compiler_params=pltpu.CompilerParams(
        needs_layout_passes=False, use_tc_tiling_on_sc=True),
    scratch_types=[
        pltpu.VMEM((B,), jnp.int32),
        pltpu.VMEM((B,), jnp.int32),
        pltpu.VMEM((LCAP,), jnp.int32),
        pltpu.VMEM((LCAP,), jnp.int32),
        pltpu.VMEM((LCAP,), jnp.int32),
        pltpu.VMEM((LCAP,), jnp.int32),
        pltpu.VMEM((272,), jnp.int32),
        pltpu.VMEM((272,), jnp.int32),
        pltpu.VMEM((2, D, SB), jnp.float32),
        pltpu.VMEM((D * TAIL_U,), jnp.float32),
        pltpu.VMEM((D * TAIL_I,), jnp.float32),
        pltpu.VMEM((32, D), jnp.float32),
        pltpu.SemaphoreType.DMA,
        pltpu.SemaphoreType.DMA,
        pltpu.SemaphoreType.DMA,
    ],
)(_extract_body)


def _compute_body(uflat_hbm, iflat_hbm, w_hbm, b_hbm, out_hbm,
                  urows, irows, w_v, b_v, out_v):
    c = lax.axis_index("c")
    s = lax.axis_index("s")
    wid = s * NC + c
    base = wid * BPW

    pltpu.sync_copy(uflat_hbm.at[pl.ds(base * D, BPW * D)], urows)
    pltpu.sync_copy(iflat_hbm.at[pl.ds(base * D, BPW * D)], irows)
    pltpu.sync_copy(w_hbm, w_v)
    pltpu.sync_copy(b_hbm, b_v)

    w_chunks = [w_v[pl.ds(k * 16, 16)] for k in range(4)]
    lane = lax.iota(jnp.int32, 16)
    m15 = lane == 15
    lane_consts = [jnp.full((16,), r, jnp.int32) for r in range(16)]

    def group_body(g, carry):
        rb = g * 16
        view = out_v.at[pl.ds(rb, 16)]
        for r in range(16):
            row64 = (rb + r) * D
            acc = (urows[pl.ds(row64, 16)] * irows[pl.ds(row64, 16)]
                   * w_chunks[0])
            for k in range(1, 4):
                acc = acc + (urows[pl.ds(row64 + k * 16, 16)]
                             * irows[pl.ds(row64 + k * 16, 16)]
                             * w_chunks[k])
            cum = plsc.cumsum(acc)
            plsc.store_scatter(view, [lane_consts[r]], cum, mask=m15)
        return carry

    lax.fori_loop(0, BPW // 16, group_body, 0)

    b_vec = b_v[...]

    def sig_body(k, carry):
        v = out_v[pl.ds(k * 16, 16)] + b_vec
        out_v[pl.ds(k * 16, 16)] = 1.0 / (1.0 + jnp.exp(-v))
        return carry

    lax.fori_loop(0, BPW // 16, sig_body, 0)
    pltpu.sync_copy(out_v, out_hbm.at[pl.ds(base, BPW)])


_compute_call = functools.partial(
    pl.kernel,
    out_type=jax.ShapeDtypeStruct((B,), jnp.float32),
    mesh=_mesh,
    compiler_params=pltpu.CompilerParams(
        needs_layout_passes=False, use_tc_tiling_on_sc=True),
    scratch_types=[
        pltpu.VMEM((BPW * D,), jnp.float32),
        pltpu.VMEM((BPW * D,), jnp.float32),
        pltpu.VMEM((D,), jnp.float32),
        pltpu.VMEM((16,), jnp.float32),
        pltpu.VMEM((BPW,), jnp.float32),
    ],
)(_compute_body)


@jax.jit
def kernel(user_indices, item_indices, user_table, item_table, W, b):
    ui = user_indices.astype(jnp.int32)
    ii = item_indices.astype(jnp.int32)
    ut_t = user_table.T
    it_t = item_table.T
    ut_tail = user_table[NSB_U * SB:].T.reshape(-1)
    it_tail = item_table[NSB_I * SB:].T.reshape(-1)
    wf = jnp.reshape(W, (D,)).astype(jnp.float32)
    bb = jnp.broadcast_to(jnp.reshape(b, (1,)), (16,)).astype(jnp.float32)
    uflat, iflat = _extract_call(ui, ii, ut_t, it_t, ut_tail, it_tail)
    out = _compute_call(uflat, iflat, wf, bb)
    return out.reshape(B, 1)
